# baseline (speedup 1.0000x reference)
import jax
import jax.numpy as jnp
from jax.experimental import pallas as pl
from jax.experimental.pallas import tpu as pltpu

_Q = 1024
_D = 768
_K = 100000
_BK = 5120
_NC = 2
_NJ = 10
_HALF = _BK // 4
_CPH = _HALF // 128
_NEG = -3.4e38
_IMAX = 2147483647


def _fold(s, chunk0, nchunks, R, T, first_masked_lanes=None):
    lane = jax.lax.broadcasted_iota(jnp.int32, (_Q, 128), 1)
    for c in range(nchunks):
        sc = jax.lax.slice_in_dim(s, c * 128, (c + 1) * 128, axis=1)
        if first_masked_lanes is not None and c == nchunks - 1:
            sc = jnp.where(lane < first_masked_lanes, sc, _NEG)
        upd = sc > R
        R = jnp.where(upd, sc, R)
        T = jnp.where(upd, jnp.int32(chunk0 + c), T)
    return R, T


def _topk_kernel(q_ref, k_ref, Rout_ref, Gout_ref, R_ref, T_ref):
    c = pl.program_id(0)
    j = pl.program_id(1)
    b = c * _NJ + j

    @pl.when(j == 0)
    def _init():
        R_ref[...] = jnp.full((_Q, 128), _NEG, jnp.float32)
        T_ref[...] = jnp.zeros((_Q, 128), jnp.int32)

    @pl.when(b < _NC * _NJ - 1)
    def _full_block():
        R = R_ref[...]
        T = T_ref[...]
        for h in range(4):
            kh = k_ref[h * _HALF:(h + 1) * _HALF, :]
            s = jax.lax.dot_general(
                q_ref[...], kh,
                dimension_numbers=(((1,), (1,)), ((), ())),
                preferred_element_type=jnp.float32,
            )
            R, T = _fold(s, b * (_BK // 128) + h * _CPH, _CPH, R, T)
        R_ref[...] = R
        T_ref[...] = T

    @pl.when(b == _NC * _NJ - 1)
    def _tail_block():
        valid = _K - (_NC * _NJ - 1) * _BK
        R = R_ref[...]
        T = T_ref[...]
        for h in range(4):
            hvalid = min(max(valid - h * _HALF, 0), _HALF)
            if hvalid == 0:
                continue
            vchunks = hvalid // 128
            rag = hvalid - vchunks * 128
            kh = k_ref[h * _HALF:(h + 1) * _HALF, :]
            s = jax.lax.dot_general(
                q_ref[...], kh,
                dimension_numbers=(((1,), (1,)), ((), ())),
                preferred_element_type=jnp.float32,
            )
            chunk0 = b * (_BK // 128) + h * _CPH
            if vchunks:
                R, T = _fold(s, chunk0, vchunks, R, T)
            if rag:
                R, T = _fold(
                    jax.lax.slice_in_dim(s, vchunks * 128,
                                         (vchunks + 1) * 128, axis=1),
                    chunk0 + vchunks, 1, R, T, first_masked_lanes=rag)
        R_ref[...] = R
        T_ref[...] = T

    @pl.when(j == _NJ - 1)
    def _emit():
        lane = jax.lax.broadcasted_iota(jnp.int32, (_Q, 128), 1)
        Rout_ref[...] = R_ref[...]
        Gout_ref[...] = T_ref[...] * 128 + lane


def _merge_kernel(R_ref, G_ref, val_ref, idx_ref):
    R = R_ref[...]
    G = G_ref[...]
    v = jnp.max(R, axis=1, keepdims=True)
    idxv = jnp.min(jnp.where(R == v, G, _IMAX), axis=1, keepdims=True)
    val_ref[...] = v
    idx_ref[...] = idxv


def kernel(queries, keys):
    Rp, Gp = pl.pallas_call(
        _topk_kernel,
        grid=(_NC, _NJ),
        in_specs=[
            pl.BlockSpec((_Q, _D), lambda c, j: (0, 0)),
            pl.BlockSpec((_BK, _D), lambda c, j: (c * _NJ + j, 0)),
        ],
        out_specs=[
            pl.BlockSpec((_Q, 128), lambda c, j: (0, c)),
            pl.BlockSpec((_Q, 128), lambda c, j: (0, c)),
        ],
        out_shape=[
            jax.ShapeDtypeStruct((_Q, _NC * 128), jnp.float32),
            jax.ShapeDtypeStruct((_Q, _NC * 128), jnp.int32),
        ],
        scratch_shapes=[
            pltpu.VMEM((_Q, 128), jnp.float32),
            pltpu.VMEM((_Q, 128), jnp.int32),
        ],
        compiler_params=pltpu.CompilerParams(
            dimension_semantics=("parallel", "arbitrary"),
        ),
    )(queries, keys)

    top_vals, top_idx = pl.pallas_call(
        _merge_kernel,
        out_shape=[
            jax.ShapeDtypeStruct((_Q, 1), jnp.float32),
            jax.ShapeDtypeStruct((_Q, 1), jnp.int32),
        ],
    )(Rp, Gp)
    return top_vals, top_idx
